# baseline (speedup 1.0000x reference)
import functools

import jax
import jax.numpy as jnp
from jax import lax
from jax.experimental import pallas as pl
from jax.experimental.pallas import tpu as pltpu
from jax.experimental.pallas import tpu_sc as plsc


def _pos_enc(seq_len, d_model):
    pos = jnp.arange(seq_len, dtype=jnp.float32)[:, None]
    _2i = jnp.arange(0, d_model, 2, dtype=jnp.float32)
    enc = jnp.zeros((seq_len, d_model), dtype=jnp.float32)
    enc = enc.at[:, 0::2].set(jnp.sin(pos / 10000 ** (_2i / d_model)))
    enc = enc.at[:, 1::2].set(jnp.cos(pos / 10000 ** (_2i / d_model)))
    return enc


@functools.lru_cache(maxsize=None)
def _build(B, S, D):
    info = plsc.get_sparse_core_info()
    NC, NS, L = info.num_cores, info.num_subcores, info.num_lanes
    NW = NC * NS
    SPT = S // NW
    CS = 16
    NCH = SPT // CS
    NB = B
    NITEM = NCH * B
    NV = D // L

    mesh = plsc.VectorSubcoreMesh(core_axis_name="c", subcore_axis_name="s")

    @functools.partial(
        pl.kernel,
        mesh=mesh,
        out_type=jax.ShapeDtypeStruct((B, S, D), jnp.float32),
        scratch_types=[
            pltpu.VMEM((B * SPT,), jnp.int32),
            pltpu.VMEM((NB, CS, D), jnp.float32),
            pltpu.VMEM((2, CS, D), jnp.float32),
            pltpu.SemaphoreType.DMA((NB,)),
            pltpu.SemaphoreType.DMA((NB,)),
            pltpu.SemaphoreType.DMA((2,)),
        ],
    )
    def embed(x_hbm, table_hbm, enc_hbm, out_hbm,
              idx_v, rbufs, ebufs, gsem, ssem, esem):
        wid = lax.axis_index("s") * NC + lax.axis_index("c")
        s0 = wid * SPT
        for b in range(B):
            pltpu.sync_copy(x_hbm.at[b, pl.ds(s0, SPT)],
                            idx_v.at[pl.ds(b * SPT, SPT)])

        def enc_start(c, e):
            pltpu.async_copy(enc_hbm.at[pl.ds(s0 + c * CS, CS)],
                             ebufs.at[e], esem.at[e])

        def gather_start(t):
            p = lax.rem(t, NB)
            c, b = lax.div(t, B), lax.rem(t, B)
            idx_sl = idx_v.at[pl.ds(b * SPT + c * CS, CS)]
            pltpu.async_copy(table_hbm.at[idx_sl], rbufs.at[p], gsem.at[p])

        enc_start(0, 0)
        enc_start(1, 1)
        gather_start(0)
        gather_start(1)

        def item_body(t, carry):
            p = lax.rem(t, NB)
            c, b = lax.div(t, B), lax.rem(t, B)
            e = lax.rem(c, 2)

            tg = t + 2
            @pl.when(tg < NITEM)
            def _():
                pg = lax.rem(tg, NB)
                @pl.when(t >= 2)
                def _():
                    pltpu.make_async_copy(rbufs.at[pg],
                                          out_hbm.at[0, pl.ds(0, CS)],
                                          ssem.at[pg]).wait()
                gather_start(tg)

            @pl.when(b == 0)
            def _():
                pltpu.make_async_copy(enc_hbm.at[pl.ds(0, CS)],
                                      ebufs.at[e], esem.at[e]).wait()

            pltpu.make_async_copy(table_hbm.at[idx_v.at[pl.ds(0, CS)]],
                                  rbufs.at[p], gsem.at[p]).wait()

            def row_body(i, c2):
                for k in range(NV):
                    sl = pl.ds(k * L, L)
                    plsc.addupdate(rbufs.at[p, i, sl], ebufs[e, i, sl])
                return c2

            lax.fori_loop(0, CS, row_body, 0)

            @pl.when((b == B - 1) & (c + 2 < NCH))
            def _():
                enc_start(c + 2, e)

            pltpu.async_copy(rbufs.at[p],
                             out_hbm.at[b, pl.ds(s0 + c * CS, CS)],
                             ssem.at[p])
            return carry

        lax.fori_loop(0, NITEM, item_body, 0)

        for p in range(NB):
            pltpu.make_async_copy(rbufs.at[p],
                                  out_hbm.at[0, pl.ds(0, CS)],
                                  ssem.at[p]).wait()

    return embed


def kernel(x, table):
    B, S = x.shape
    _, D = table.shape
    enc = _pos_enc(S, D)
    return _build(B, S, D)(x.astype(jnp.int32), table, enc)


# speedup vs baseline: 1.6444x; 1.6444x over previous
import functools

import jax
import jax.numpy as jnp
import numpy as np
from jax import lax
from jax.experimental import pallas as pl
from jax.experimental.pallas import tpu as pltpu
from jax.experimental.pallas import tpu_sc as plsc


@functools.lru_cache(maxsize=None)
def _pos_enc(seq_len, d_model):
    pos = np.arange(seq_len, dtype=np.float32)[:, None]
    _2i = np.arange(0, d_model, 2, dtype=np.float32)
    enc = np.zeros((seq_len, d_model), dtype=np.float32)
    enc[:, 0::2] = np.sin(pos / 10000 ** (_2i / np.float32(d_model)))
    enc[:, 1::2] = np.cos(pos / 10000 ** (_2i / np.float32(d_model)))
    return enc


@functools.lru_cache(maxsize=None)
def _build(B, S, D):
    info = plsc.get_sparse_core_info()
    NC, NS, L = info.num_cores, info.num_subcores, info.num_lanes
    NW = NC * NS
    SPT = S // NW
    CS = 32
    NCH = SPT // CS
    NB = 3
    NITEM = NCH * B
    NV = D // L

    mesh = plsc.VectorSubcoreMesh(core_axis_name="c", subcore_axis_name="s")

    @functools.partial(
        pl.kernel,
        mesh=mesh,
        out_type=jax.ShapeDtypeStruct((B, S, D), jnp.float32),
        scratch_types=[
            pltpu.VMEM((B * SPT,), jnp.int32),
            pltpu.VMEM((NB, CS, D), jnp.float32),
            pltpu.VMEM((2, CS, D), jnp.float32),
            pltpu.SemaphoreType.DMA((NB,)),
            pltpu.SemaphoreType.DMA((NB,)),
            pltpu.SemaphoreType.DMA((2,)),
        ],
    )
    def embed(x_hbm, table_hbm, enc_hbm, out_hbm,
              idx_v, rbufs, ebufs, gsem, ssem, esem):
        wid = lax.axis_index("s") * NC + lax.axis_index("c")
        s0 = wid * SPT
        for b in range(B):
            pltpu.sync_copy(x_hbm.at[b, pl.ds(s0, SPT)],
                            idx_v.at[pl.ds(b * SPT, SPT)])

        def enc_start(c, e):
            pltpu.async_copy(enc_hbm.at[pl.ds(s0 + c * CS, CS)],
                             ebufs.at[e], esem.at[e])

        def gather_start(t):
            p = lax.rem(t, NB)
            c, b = lax.div(t, B), lax.rem(t, B)
            idx_sl = idx_v.at[pl.ds(b * SPT + c * CS, CS)]
            pltpu.async_copy(table_hbm.at[idx_sl], rbufs.at[p], gsem.at[p])

        enc_start(0, 0)
        enc_start(1, 1)
        gather_start(0)

        def item_body(t, carry):
            p = lax.rem(t, NB)
            c, b = lax.div(t, B), lax.rem(t, B)
            e = lax.rem(c, 2)

            tg = t + 1
            @pl.when(tg < NITEM)
            def _():
                pg = lax.rem(tg, NB)
                @pl.when(t >= 2)
                def _():
                    pltpu.make_async_copy(rbufs.at[pg],
                                          out_hbm.at[0, pl.ds(0, CS)],
                                          ssem.at[pg]).wait()
                gather_start(tg)

            @pl.when(b == 0)
            def _():
                pltpu.make_async_copy(enc_hbm.at[pl.ds(0, CS)],
                                      ebufs.at[e], esem.at[e]).wait()

            pltpu.make_async_copy(table_hbm.at[idx_v.at[pl.ds(0, CS)]],
                                  rbufs.at[p], gsem.at[p]).wait()

            def row_body(i, c2):
                for k in range(NV):
                    sl = pl.ds(k * L, L)
                    plsc.addupdate(rbufs.at[p, i, sl], ebufs[e, i, sl])
                return c2

            lax.fori_loop(0, CS, row_body, 0)

            @pl.when((b == B - 1) & (c + 2 < NCH))
            def _():
                enc_start(c + 2, e)

            pltpu.async_copy(rbufs.at[p],
                             out_hbm.at[b, pl.ds(s0 + c * CS, CS)],
                             ssem.at[p])
            return carry

        lax.fori_loop(0, NITEM, item_body, 0)

        for p in range(NB):
            pltpu.make_async_copy(rbufs.at[p],
                                  out_hbm.at[0, pl.ds(0, CS)],
                                  ssem.at[p]).wait()

    return embed


def kernel(x, table):
    B, S = x.shape
    _, D = table.shape
    enc = _pos_enc(S, D)
    return _build(B, S, D)(x.astype(jnp.int32), table, enc)


# speedup vs baseline: 2.8440x; 1.7295x over previous
import functools

import jax
import jax.numpy as jnp
import numpy as np
from jax import lax
from jax.experimental import pallas as pl
from jax.experimental.pallas import tpu as pltpu
from jax.experimental.pallas import tpu_sc as plsc


@functools.lru_cache(maxsize=None)
def _pos_enc(seq_len, d_model):
    pos = np.arange(seq_len, dtype=np.float32)[:, None]
    _2i = np.arange(0, d_model, 2, dtype=np.float32)
    enc = np.zeros((seq_len, d_model), dtype=np.float32)
    enc[:, 0::2] = np.sin(pos / 10000 ** (_2i / np.float32(d_model)))
    enc[:, 1::2] = np.cos(pos / 10000 ** (_2i / np.float32(d_model)))
    return enc


@functools.lru_cache(maxsize=None)
def _build(B, S, D):
    info = plsc.get_sparse_core_info()
    NC, NS, L = info.num_cores, info.num_subcores, info.num_lanes
    NW = NC * NS
    SPT = S // NW
    CS = 32
    NCH = SPT // CS
    NB = 3
    NITEM = NCH * B
    NV = D // L

    mesh = plsc.VectorSubcoreMesh(core_axis_name="c", subcore_axis_name="s")

    @functools.partial(
        pl.kernel,
        mesh=mesh,
        out_type=jax.ShapeDtypeStruct((B, S, D), jnp.float32),
        scratch_types=[
            pltpu.VMEM((B * SPT,), jnp.int32),
            pltpu.VMEM((NB, CS, D), jnp.float32),
            pltpu.VMEM((2, CS, D), jnp.float32),
            pltpu.SemaphoreType.DMA((NB,)),
            pltpu.SemaphoreType.DMA((NB,)),
            pltpu.SemaphoreType.DMA((2,)),
        ],
    )
    def embed(x_hbm, table_hbm, enc_hbm, out_hbm,
              idx_v, rbufs, ebufs, gsem, ssem, esem):
        wid = lax.axis_index("s") * NC + lax.axis_index("c")
        s0 = wid * SPT
        for b in range(B):
            pltpu.sync_copy(x_hbm.at[b, pl.ds(s0, SPT)],
                            idx_v.at[pl.ds(b * SPT, SPT)])

        def enc_start(c, e):
            pltpu.async_copy(enc_hbm.at[pl.ds(s0 + c * CS, CS)],
                             ebufs.at[e], esem.at[e])

        def gather_start(t):
            p = lax.rem(t, NB)
            c, b = lax.div(t, B), lax.rem(t, B)
            idx_sl = idx_v.at[pl.ds(b * SPT + c * CS, CS)]
            pltpu.async_copy(table_hbm.at[idx_sl], rbufs.at[p], gsem.at[p])

        enc_start(0, 0)
        enc_start(1, 1)
        gather_start(0)

        def item_body(t, carry):
            p = lax.rem(t, NB)
            c, b = lax.div(t, B), lax.rem(t, B)
            e = lax.rem(c, 2)

            tg = t + 1
            @pl.when(tg < NITEM)
            def _():
                pg = lax.rem(tg, NB)
                @pl.when(t >= 2)
                def _():
                    pltpu.make_async_copy(rbufs.at[pg],
                                          out_hbm.at[0, pl.ds(0, CS)],
                                          ssem.at[pg]).wait()
                gather_start(tg)

            @pl.when(b == 0)
            def _():
                pltpu.make_async_copy(enc_hbm.at[pl.ds(0, CS)],
                                      ebufs.at[e], esem.at[e]).wait()

            pltpu.make_async_copy(table_hbm.at[idx_v.at[pl.ds(0, CS)]],
                                  rbufs.at[p], gsem.at[p]).wait()

            @plsc.parallel_loop(0, CS, step=1, unroll=2)
            def _(i):
                for k in range(NV):
                    sl = pl.ds(k * L, L)
                    plsc.addupdate(rbufs.at[p, i, sl], ebufs[e, i, sl])

            @pl.when((b == B - 1) & (c + 2 < NCH))
            def _():
                enc_start(c + 2, e)

            pltpu.async_copy(rbufs.at[p],
                             out_hbm.at[b, pl.ds(s0 + c * CS, CS)],
                             ssem.at[p])
            return carry

        lax.fori_loop(0, NITEM, item_body, 0)

        for p in range(NB):
            pltpu.make_async_copy(rbufs.at[p],
                                  out_hbm.at[0, pl.ds(0, CS)],
                                  ssem.at[p]).wait()

    return embed


def kernel(x, table):
    B, S = x.shape
    _, D = table.shape
    enc = _pos_enc(S, D)
    return _build(B, S, D)(x.astype(jnp.int32), table, enc)
